# baseline (speedup 1.0000x reference)
import functools

import jax
import jax.numpy as jnp
from jax import lax
from jax.experimental import pallas as pl
from jax.experimental.pallas import tpu as pltpu
from jax.experimental.pallas import tpu_sc as plsc

D = 32
_W = 128


def _mm(a, b, precision=lax.Precision.HIGHEST):
    return lax.dot_general(a, b, (((1,), (0,)), ((), ())),
                           precision=precision,
                           preferred_element_type=jnp.float32)


def _sc_mesh():
    return plsc.VectorSubcoreMesh(core_axis_name="c", subcore_axis_name="s")


def _sc_gather(table, idx):
    e = idx.shape[0]
    d = table.shape[1]
    idx2 = idx.reshape(1, e)

    @functools.partial(
        pl.kernel,
        out_type=jax.ShapeDtypeStruct((e, d), table.dtype),
        mesh=_sc_mesh(),
        compiler_params=pltpu.CompilerParams(use_tc_tiling_on_sc=False),
    )
    def k(tab_hbm, i_hbm, o_hbm):
        def body(i_vmem, o_vmem):
            pltpu.sync_copy(tab_hbm.at[i_vmem.at[0]], o_vmem)

        pltpu.emit_pipeline(
            body,
            grid=(e // _W,),
            in_specs=[pl.BlockSpec((1, _W), lambda i: (0, i))],
            out_specs=[pl.BlockSpec((_W, d), lambda i: (i, 0))],
            core_axis_name=("c", "s"),
            dimension_semantics=(pltpu.PARALLEL,),
        )(i_hbm, o_hbm)

    return k(table, idx2)


def _sc_scatter_add(vals, idx, n_rows, zeros):
    e, d = vals.shape
    idx2 = idx.reshape(1, e)
    per_s = n_rows // 16

    @functools.partial(
        pl.kernel,
        out_type=jax.ShapeDtypeStruct((2, n_rows, d), vals.dtype),
        mesh=_sc_mesh(),
        scratch_types=[pltpu.VMEM_SHARED((n_rows, d), jnp.float32)],
        compiler_params=pltpu.CompilerParams(use_tc_tiling_on_sc=False),
    )
    def k(v_hbm, i_hbm, z_hbm, o_hbm, acc_sh):
        cid = lax.axis_index("c")
        sid = lax.axis_index("s")
        sl = pl.ds(sid * per_s, per_s)
        pltpu.sync_copy(z_hbm.at[sl], acc_sh.at[sl])
        plsc.subcore_barrier()

        def body(v_vmem, i_vmem):
            pltpu.sync_copy(v_vmem, acc_sh.at[i_vmem.at[0]], add=True)

        pltpu.emit_pipeline(
            body,
            grid=(e // _W,),
            in_specs=[pl.BlockSpec((_W, d), lambda i: (i, 0)),
                      pl.BlockSpec((1, _W), lambda i: (0, i))],
            out_specs=[],
            core_axis_name=("c", "s"),
            dimension_semantics=(pltpu.PARALLEL,),
        )(v_hbm, i_hbm)
        plsc.subcore_barrier()
        pltpu.sync_copy(acc_sh.at[sl], o_hbm.at[cid].at[sl])

    return k(vals, idx2, zeros)


def _full(spec_arr):
    nd = spec_arr.ndim
    return pl.BlockSpec(spec_arr.shape, lambda i: (0,) * nd)


def _node_tc(x, W_node, b_node, bn=2000):
    n = x.shape[0]

    def body(x_ref, w_ref, b_ref, o_ref):
        o_ref[...] = jnp.maximum(
            _mm(x_ref[...], w_ref[...], lax.Precision.DEFAULT) + b_ref[...],
            0.0)

    return pl.pallas_call(
        body,
        grid=(n // bn,),
        in_specs=[pl.BlockSpec((bn, x.shape[1]), lambda i: (i, 0)),
                  _full(W_node), _full(b_node)],
        out_specs=pl.BlockSpec((bn, D), lambda i: (i, 0)),
        out_shape=jax.ShapeDtypeStruct((n, D), jnp.float32),
    )(x, W_node, b_node)


def _msg_tc(edge_attr, g, W_ea, b_ea, W_nn1, b_nn1, W_nn2, b_nn2, rep,
            be=1600):
    e = edge_attr.shape[0]

    def body(ea_ref, g_ref, wea, bea, w1, b1, w2, b2, rep_ref, o_ref):
        fast = lax.Precision.DEFAULT
        ea = jnp.maximum(_mm(ea_ref[...], wea[...], fast) + bea[...], 0.0)
        r = jnp.maximum(_mm(ea, w1[...], fast) + b1[...], 0.0)
        w = _mm(r, w2[...], fast) + b2[...]
        gexp = _mm(g_ref[...], rep_ref[...], fast)
        t = gexp * w
        a = t[:, :512] + t[:, 512:]
        a = a[:, :256] + a[:, 256:]
        a = a[:, :128] + a[:, 128:]
        a = a[:, :64] + a[:, 64:]
        o_ref[...] = a[:, :32] + a[:, 32:]

    return pl.pallas_call(
        body,
        grid=(e // be,),
        in_specs=[pl.BlockSpec((be, edge_attr.shape[1]), lambda i: (i, 0)),
                  pl.BlockSpec((be, D), lambda i: (i, 0)),
                  _full(W_ea), _full(b_ea), _full(W_nn1), _full(b_nn1),
                  _full(W_nn2), _full(b_nn2), _full(rep)],
        out_specs=pl.BlockSpec((be, D), lambda i: (i, 0)),
        out_shape=jax.ShapeDtypeStruct((e, D), jnp.float32),
    )(edge_attr, g, W_ea, b_ea, W_nn1, b_nn1, W_nn2, b_nn2, rep)


def _gru_tc(a0, a1, a2, a3, d0, d1, h, b_conv, W_ih, b_ih, W_hh, b_hh,
            bn=2000):
    n = h.shape[0]

    def body(a0r, a1r, a2r, a3r, d0r, d1r, hr, bc, wih, bih, whh, bhh, o_ref):
        deg = jnp.maximum(d0r[...][:, 0:1] + d1r[...][:, 0:1], 1.0)
        agg = ((a0r[...] + a1r[...]) + (a2r[...] + a3r[...])) / deg
        m = jnp.maximum(agg + bc[...], 0.0)
        fast = lax.Precision.DEFAULT
        gi = _mm(m, wih[...], fast) + bih[...]
        gh = _mm(hr[...], whh[...], fast) + bhh[...]
        r = jax.nn.sigmoid(gi[:, :D] + gh[:, :D])
        z = jax.nn.sigmoid(gi[:, D:2 * D] + gh[:, D:2 * D])
        nn = jnp.tanh(gi[:, 2 * D:] + r * gh[:, 2 * D:])
        o_ref[...] = (1.0 - z) * nn + z * hr[...]

    return pl.pallas_call(
        body,
        grid=(n // bn,),
        in_specs=[pl.BlockSpec((bn, D), lambda i: (i, 0)),
                  pl.BlockSpec((bn, D), lambda i: (i, 0)),
                  pl.BlockSpec((bn, D), lambda i: (i, 0)),
                  pl.BlockSpec((bn, D), lambda i: (i, 0)),
                  pl.BlockSpec((bn, 16), lambda i: (i, 0)),
                  pl.BlockSpec((bn, 16), lambda i: (i, 0)),
                  pl.BlockSpec((bn, D), lambda i: (i, 0)),
                  _full(b_conv), _full(W_ih), _full(b_ih), _full(W_hh),
                  _full(b_hh)],
        out_specs=pl.BlockSpec((bn, D), lambda i: (i, 0)),
        out_shape=jax.ShapeDtypeStruct((n, D), jnp.float32),
    )(a0, a1, a2, a3, d0, d1, h, b_conv, W_ih, b_ih, W_hh, b_hh)


def _feat(t0, t1):
    return jnp.concatenate([(t0 + t1) * 0.5, t0 * t1, (t0 - t1) ** 2], axis=1)


def _readout_stats_tc(t01, e, be=4000):
    nb = e // be

    def body(t0r, t1r, s_ref, q_ref):
        f = _feat(t0r[...], t1r[...])

        @pl.when(pl.program_id(0) == 0)
        def _():
            s_ref[...] = jnp.zeros_like(s_ref)
            q_ref[...] = jnp.zeros_like(q_ref)

        s_ref[...] += jnp.sum(f, axis=0, keepdims=True)
        q_ref[...] += jnp.sum(f * f, axis=0, keepdims=True)

    return pl.pallas_call(
        body,
        grid=(nb,),
        in_specs=[pl.BlockSpec((be, D), lambda i: (i, 0)),
                  pl.BlockSpec((be, D), lambda i: (i + nb, 0))],
        out_specs=[pl.BlockSpec((1, 3 * D), lambda i: (0, 0)),
                   pl.BlockSpec((1, 3 * D), lambda i: (0, 0))],
        out_shape=[jax.ShapeDtypeStruct((1, 3 * D), jnp.float32),
                   jax.ShapeDtypeStruct((1, 3 * D), jnp.float32)],
    )(t01, t01)


def _readout_tc(t01, ea3, s, q, gamma, beta, W_w, W_b, be=4000):
    e = ea3.shape[0]
    nb = e // be
    inv_e = 1.0 / float(e)

    def body(t0r, t1r, ea3r, s_ref, q_ref, g_ref, b_ref, ww, wb, o_ref):
        f = _feat(t0r[...], t1r[...])
        mean = s_ref[...] * inv_e
        var = q_ref[...] * inv_e - mean * mean
        scale = g_ref[...] * lax.rsqrt(var + 1e-5)
        shift = b_ref[...] - mean * scale
        wgt = _mm(ea3r[...], ww[...], lax.Precision.DEFAULT)
        bias = _mm(ea3r[...], wb[...], lax.Precision.DEFAULT)
        fn = f * scale + shift
        o_ref[...] = jnp.sum(fn * wgt, axis=1, keepdims=True) + bias

    return pl.pallas_call(
        body,
        grid=(nb,),
        in_specs=[pl.BlockSpec((be, D), lambda i: (i, 0)),
                  pl.BlockSpec((be, D), lambda i: (i + nb, 0)),
                  pl.BlockSpec((be, ea3.shape[1]), lambda i: (i, 0)),
                  _full(s), _full(q), _full(gamma), _full(beta),
                  _full(W_w), _full(W_b)],
        out_specs=pl.BlockSpec((be, 1), lambda i: (i, 0)),
        out_shape=jax.ShapeDtypeStruct((e, 1), jnp.float32),
    )(t01, t01, ea3, s, q, gamma, beta, W_w, W_b)


def kernel(x, edge_attr, edge_attr3, W_node, b_node, W_ea, b_ea, W_nn1, b_nn1,
           W_nn2, b_nn2, b_conv, W_ih, b_ih, W_hh, b_hh, gamma, beta, W_w, W_b,
           edge_index, edge_index3):
    n = x.shape[0]
    e = edge_index.shape[1]
    e3 = edge_index3.shape[1]

    src = edge_index[0].astype(jnp.int32)
    dst = edge_index[1].astype(jnp.int32)
    i3a = edge_index3[0].astype(jnp.int32)
    i3b = edge_index3[1].astype(jnp.int32)

    b_node2 = b_node.reshape(1, -1)
    b_ea2 = b_ea.reshape(1, -1)
    b_nn12 = b_nn1.reshape(1, -1)
    b_nn22 = b_nn2.reshape(1, -1)
    b_conv2 = b_conv.reshape(1, -1)
    b_ih2 = b_ih.reshape(1, -1)
    b_hh2 = b_hh.reshape(1, -1)
    gamma2 = gamma.reshape(1, -1)
    beta2 = beta.reshape(1, -1)

    ll = jnp.arange(D * D)
    rep = (ll[None, :] // D == jnp.arange(D)[:, None]).astype(jnp.float32)

    zeros_nd = jnp.zeros((n, D), jnp.float32)
    zeros_n16 = jnp.zeros((n, 16), jnp.float32)
    ones_e16 = jnp.ones((e, 16), jnp.float32)

    h = _node_tc(x, W_node, b_node2)

    degp = _sc_scatter_add(ones_e16, dst, n, zeros_n16)
    d0, d1 = degp[0], degp[1]

    he = e // 2
    src_c = (src[:he], src[he:])
    dst_c = (dst[:he], dst[he:])
    ea_c = (edge_attr[:he], edge_attr[he:])

    for _ in range(3):
        g0 = _sc_gather(h, src_c[0])
        g1 = _sc_gather(h, src_c[1])
        msg0 = _msg_tc(ea_c[0], g0, W_ea, b_ea2, W_nn1, b_nn12, W_nn2, b_nn22,
                       rep)
        msg1 = _msg_tc(ea_c[1], g1, W_ea, b_ea2, W_nn1, b_nn12, W_nn2, b_nn22,
                       rep)
        p0 = _sc_scatter_add(msg0, dst_c[0], n, zeros_nd)
        p1 = _sc_scatter_add(msg1, dst_c[1], n, zeros_nd)
        h = _gru_tc(p0[0], p0[1], p1[0], p1[1], d0, d1, h, b_conv2, W_ih,
                    b_ih2, W_hh, b_hh2)

    t01 = _sc_gather(h, jnp.concatenate([i3a, i3b]))
    s, q = _readout_stats_tc(t01, e3)
    yhat = _readout_tc(t01, edge_attr3, s, q, gamma2, beta2, W_w, W_b)
    return yhat.reshape(e3)


# speedup vs baseline: 1.0037x; 1.0037x over previous
import functools

import jax
import jax.numpy as jnp
from jax import lax
from jax.experimental import pallas as pl
from jax.experimental.pallas import tpu as pltpu
from jax.experimental.pallas import tpu_sc as plsc

D = 32
_W = 128


def _mm(a, b, precision=lax.Precision.HIGHEST):
    return lax.dot_general(a, b, (((1,), (0,)), ((), ())),
                           precision=precision,
                           preferred_element_type=jnp.float32)


def _sc_mesh():
    return plsc.VectorSubcoreMesh(core_axis_name="c", subcore_axis_name="s")


def _sc_gather(table, idx):
    e = idx.shape[0]
    d = table.shape[1]
    idx2 = idx.reshape(1, e)

    @functools.partial(
        pl.kernel,
        out_type=jax.ShapeDtypeStruct((e, d), table.dtype),
        mesh=_sc_mesh(),
        compiler_params=pltpu.CompilerParams(use_tc_tiling_on_sc=False),
    )
    def k(tab_hbm, i_hbm, o_hbm):
        def body(i_vmem, o_vmem):
            pltpu.sync_copy(tab_hbm.at[i_vmem.at[0]], o_vmem)

        pltpu.emit_pipeline(
            body,
            grid=(e // _W,),
            in_specs=[pl.BlockSpec((1, _W), lambda i: (0, i))],
            out_specs=[pl.BlockSpec((_W, d), lambda i: (i, 0))],
            core_axis_name=("c", "s"),
            dimension_semantics=(pltpu.PARALLEL,),
        )(i_hbm, o_hbm)

    return k(table, idx2)


def _sc_scatter_add(vals, idx, n_rows, zeros):
    e, d = vals.shape
    idx2 = idx.reshape(1, e)
    per_s = n_rows // 16

    @functools.partial(
        pl.kernel,
        out_type=jax.ShapeDtypeStruct((2, n_rows, d), vals.dtype),
        mesh=_sc_mesh(),
        scratch_types=[pltpu.VMEM_SHARED((n_rows, d), jnp.float32)],
        compiler_params=pltpu.CompilerParams(use_tc_tiling_on_sc=False),
    )
    def k(v_hbm, i_hbm, z_hbm, o_hbm, acc_sh):
        cid = lax.axis_index("c")
        sid = lax.axis_index("s")
        sl = pl.ds(sid * per_s, per_s)
        pltpu.sync_copy(z_hbm.at[sl], acc_sh.at[sl])
        plsc.subcore_barrier()

        def body(v_vmem, i_vmem):
            pltpu.sync_copy(v_vmem, acc_sh.at[i_vmem.at[0]], add=True)

        pltpu.emit_pipeline(
            body,
            grid=(e // _W,),
            in_specs=[pl.BlockSpec((_W, d), lambda i: (i, 0)),
                      pl.BlockSpec((1, _W), lambda i: (0, i))],
            out_specs=[],
            core_axis_name=("c", "s"),
            dimension_semantics=(pltpu.PARALLEL,),
        )(v_hbm, i_hbm)
        plsc.subcore_barrier()
        pltpu.sync_copy(acc_sh.at[sl], o_hbm.at[cid].at[sl])

    return k(vals, idx2, zeros)


def _sc_scatter_add2(v0, v1, idx0, idx1, n_rows, zeros):
    e, d = v0.shape
    i0 = idx0.reshape(1, e)
    i1 = idx1.reshape(1, e)
    per_s = n_rows // 16

    @functools.partial(
        pl.kernel,
        out_type=jax.ShapeDtypeStruct((2, n_rows, d), v0.dtype),
        mesh=_sc_mesh(),
        scratch_types=[pltpu.VMEM_SHARED((n_rows, d), jnp.float32)],
        compiler_params=pltpu.CompilerParams(use_tc_tiling_on_sc=False),
    )
    def k(v0_hbm, v1_hbm, i0_hbm, i1_hbm, z_hbm, o_hbm, acc_sh):
        cid = lax.axis_index("c")
        sid = lax.axis_index("s")
        sl = pl.ds(sid * per_s, per_s)
        pltpu.sync_copy(z_hbm.at[sl], acc_sh.at[sl])
        plsc.subcore_barrier()

        def body(v_vmem, i_vmem):
            pltpu.sync_copy(v_vmem, acc_sh.at[i_vmem.at[0]], add=True)

        for v_hbm, i_hbm in ((v0_hbm, i0_hbm), (v1_hbm, i1_hbm)):
            pltpu.emit_pipeline(
                body,
                grid=(e // _W,),
                in_specs=[pl.BlockSpec((_W, d), lambda i: (i, 0)),
                          pl.BlockSpec((1, _W), lambda i: (0, i))],
                out_specs=[],
                core_axis_name=("c", "s"),
                dimension_semantics=(pltpu.PARALLEL,),
            )(v_hbm, i_hbm)
        plsc.subcore_barrier()
        pltpu.sync_copy(acc_sh.at[sl], o_hbm.at[cid].at[sl])

    return k(v0, v1, i0, i1, zeros)


def _full(spec_arr):
    nd = spec_arr.ndim
    return pl.BlockSpec(spec_arr.shape, lambda i: (0,) * nd)


def _node_tc(x, W_node, b_node, bn=2000):
    n = x.shape[0]

    def body(x_ref, w_ref, b_ref, o_ref):
        o_ref[...] = jnp.maximum(
            _mm(x_ref[...], w_ref[...], lax.Precision.DEFAULT) + b_ref[...],
            0.0)

    return pl.pallas_call(
        body,
        grid=(n // bn,),
        in_specs=[pl.BlockSpec((bn, x.shape[1]), lambda i: (i, 0)),
                  _full(W_node), _full(b_node)],
        out_specs=pl.BlockSpec((bn, D), lambda i: (i, 0)),
        out_shape=jax.ShapeDtypeStruct((n, D), jnp.float32),
    )(x, W_node, b_node)


def _msg_tc(edge_attr, g, W_ea, b_ea, W_nn1, b_nn1, W_nn2, b_nn2, rep,
            be=1600):
    e = edge_attr.shape[0]

    def body(ea_ref, g_ref, wea, bea, w1, b1, w2, b2, rep_ref, o_ref):
        fast = lax.Precision.DEFAULT
        ea = jnp.maximum(_mm(ea_ref[...], wea[...], fast) + bea[...], 0.0)
        r = jnp.maximum(_mm(ea, w1[...], fast) + b1[...], 0.0)
        w = _mm(r, w2[...], fast) + b2[...]
        gexp = _mm(g_ref[...], rep_ref[...], fast)
        t = gexp * w
        a = t[:, :512] + t[:, 512:]
        a = a[:, :256] + a[:, 256:]
        a = a[:, :128] + a[:, 128:]
        a = a[:, :64] + a[:, 64:]
        o_ref[...] = a[:, :32] + a[:, 32:]

    return pl.pallas_call(
        body,
        grid=(e // be,),
        in_specs=[pl.BlockSpec((be, edge_attr.shape[1]), lambda i: (i, 0)),
                  pl.BlockSpec((be, D), lambda i: (i, 0)),
                  _full(W_ea), _full(b_ea), _full(W_nn1), _full(b_nn1),
                  _full(W_nn2), _full(b_nn2), _full(rep)],
        out_specs=pl.BlockSpec((be, D), lambda i: (i, 0)),
        out_shape=jax.ShapeDtypeStruct((e, D), jnp.float32),
    )(edge_attr, g, W_ea, b_ea, W_nn1, b_nn1, W_nn2, b_nn2, rep)


def _gru_tc(a0, a1, d0, d1, h, b_conv, W_ih, b_ih, W_hh, b_hh, bn=2000):
    n = h.shape[0]

    def body(a0r, a1r, d0r, d1r, hr, bc, wih, bih, whh, bhh, o_ref):
        deg = jnp.maximum(d0r[...][:, 0:1] + d1r[...][:, 0:1], 1.0)
        agg = (a0r[...] + a1r[...]) / deg
        m = jnp.maximum(agg + bc[...], 0.0)
        fast = lax.Precision.DEFAULT
        gi = _mm(m, wih[...], fast) + bih[...]
        gh = _mm(hr[...], whh[...], fast) + bhh[...]
        r = jax.nn.sigmoid(gi[:, :D] + gh[:, :D])
        z = jax.nn.sigmoid(gi[:, D:2 * D] + gh[:, D:2 * D])
        nn = jnp.tanh(gi[:, 2 * D:] + r * gh[:, 2 * D:])
        o_ref[...] = (1.0 - z) * nn + z * hr[...]

    return pl.pallas_call(
        body,
        grid=(n // bn,),
        in_specs=[pl.BlockSpec((bn, D), lambda i: (i, 0)),
                  pl.BlockSpec((bn, D), lambda i: (i, 0)),
                  pl.BlockSpec((bn, 16), lambda i: (i, 0)),
                  pl.BlockSpec((bn, 16), lambda i: (i, 0)),
                  pl.BlockSpec((bn, D), lambda i: (i, 0)),
                  _full(b_conv), _full(W_ih), _full(b_ih), _full(W_hh),
                  _full(b_hh)],
        out_specs=pl.BlockSpec((bn, D), lambda i: (i, 0)),
        out_shape=jax.ShapeDtypeStruct((n, D), jnp.float32),
    )(a0, a1, d0, d1, h, b_conv, W_ih, b_ih, W_hh, b_hh)


def _feat(t0, t1):
    return jnp.concatenate([(t0 + t1) * 0.5, t0 * t1, (t0 - t1) ** 2], axis=1)


def _readout_stats_tc(t01, e, be=4000):
    nb = e // be

    def body(t0r, t1r, s_ref, q_ref):
        f = _feat(t0r[...], t1r[...])

        @pl.when(pl.program_id(0) == 0)
        def _():
            s_ref[...] = jnp.zeros_like(s_ref)
            q_ref[...] = jnp.zeros_like(q_ref)

        s_ref[...] += jnp.sum(f, axis=0, keepdims=True)
        q_ref[...] += jnp.sum(f * f, axis=0, keepdims=True)

    return pl.pallas_call(
        body,
        grid=(nb,),
        in_specs=[pl.BlockSpec((be, D), lambda i: (i, 0)),
                  pl.BlockSpec((be, D), lambda i: (i + nb, 0))],
        out_specs=[pl.BlockSpec((1, 3 * D), lambda i: (0, 0)),
                   pl.BlockSpec((1, 3 * D), lambda i: (0, 0))],
        out_shape=[jax.ShapeDtypeStruct((1, 3 * D), jnp.float32),
                   jax.ShapeDtypeStruct((1, 3 * D), jnp.float32)],
    )(t01, t01)


def _readout_tc(t01, ea3, s, q, gamma, beta, W_w, W_b, be=4000):
    e = ea3.shape[0]
    nb = e // be
    inv_e = 1.0 / float(e)

    def body(t0r, t1r, ea3r, s_ref, q_ref, g_ref, b_ref, ww, wb, o_ref):
        f = _feat(t0r[...], t1r[...])
        mean = s_ref[...] * inv_e
        var = q_ref[...] * inv_e - mean * mean
        scale = g_ref[...] * lax.rsqrt(var + 1e-5)
        shift = b_ref[...] - mean * scale
        wgt = _mm(ea3r[...], ww[...], lax.Precision.DEFAULT)
        bias = _mm(ea3r[...], wb[...], lax.Precision.DEFAULT)
        fn = f * scale + shift
        o_ref[...] = jnp.sum(fn * wgt, axis=1, keepdims=True) + bias

    return pl.pallas_call(
        body,
        grid=(nb,),
        in_specs=[pl.BlockSpec((be, D), lambda i: (i, 0)),
                  pl.BlockSpec((be, D), lambda i: (i + nb, 0)),
                  pl.BlockSpec((be, ea3.shape[1]), lambda i: (i, 0)),
                  _full(s), _full(q), _full(gamma), _full(beta),
                  _full(W_w), _full(W_b)],
        out_specs=pl.BlockSpec((be, 1), lambda i: (i, 0)),
        out_shape=jax.ShapeDtypeStruct((e, 1), jnp.float32),
    )(t01, t01, ea3, s, q, gamma, beta, W_w, W_b)


def kernel(x, edge_attr, edge_attr3, W_node, b_node, W_ea, b_ea, W_nn1, b_nn1,
           W_nn2, b_nn2, b_conv, W_ih, b_ih, W_hh, b_hh, gamma, beta, W_w, W_b,
           edge_index, edge_index3):
    n = x.shape[0]
    e = edge_index.shape[1]
    e3 = edge_index3.shape[1]

    src = edge_index[0].astype(jnp.int32)
    dst = edge_index[1].astype(jnp.int32)
    i3a = edge_index3[0].astype(jnp.int32)
    i3b = edge_index3[1].astype(jnp.int32)

    b_node2 = b_node.reshape(1, -1)
    b_ea2 = b_ea.reshape(1, -1)
    b_nn12 = b_nn1.reshape(1, -1)
    b_nn22 = b_nn2.reshape(1, -1)
    b_conv2 = b_conv.reshape(1, -1)
    b_ih2 = b_ih.reshape(1, -1)
    b_hh2 = b_hh.reshape(1, -1)
    gamma2 = gamma.reshape(1, -1)
    beta2 = beta.reshape(1, -1)

    ll = jnp.arange(D * D)
    rep = (ll[None, :] // D == jnp.arange(D)[:, None]).astype(jnp.float32)

    zeros_nd = jnp.zeros((n, D), jnp.float32)
    zeros_n16 = jnp.zeros((n, 16), jnp.float32)
    ones_e16 = jnp.ones((e, 16), jnp.float32)

    h = _node_tc(x, W_node, b_node2)

    degp = _sc_scatter_add(ones_e16, dst, n, zeros_n16)
    d0, d1 = degp[0], degp[1]

    he = e // 2
    src_c = (src[:he], src[he:])
    dst_c = (dst[:he], dst[he:])
    ea_c = (edge_attr[:he], edge_attr[he:])

    for _ in range(3):
        g0 = _sc_gather(h, src_c[0])
        g1 = _sc_gather(h, src_c[1])
        msg0 = _msg_tc(ea_c[0], g0, W_ea, b_ea2, W_nn1, b_nn12, W_nn2, b_nn22,
                       rep)
        msg1 = _msg_tc(ea_c[1], g1, W_ea, b_ea2, W_nn1, b_nn12, W_nn2, b_nn22,
                       rep)
        p = _sc_scatter_add2(msg0, msg1, dst_c[0], dst_c[1], n, zeros_nd)
        h = _gru_tc(p[0], p[1], d0, d1, h, b_conv2, W_ih, b_ih2, W_hh, b_hh2)

    t01 = _sc_gather(h, jnp.concatenate([i3a, i3b]))
    s, q = _readout_stats_tc(t01, e3)
    yhat = _readout_tc(t01, edge_attr3, s, q, gamma2, beta2, W_w, W_b)
    return yhat.reshape(e3)


# speedup vs baseline: 1.0298x; 1.0260x over previous
import functools

import jax
import jax.numpy as jnp
from jax import lax
from jax.experimental import pallas as pl
from jax.experimental.pallas import tpu as pltpu
from jax.experimental.pallas import tpu_sc as plsc

D = 32
_W = 128


def _mm(a, b, precision=lax.Precision.HIGHEST):
    return lax.dot_general(a, b, (((1,), (0,)), ((), ())),
                           precision=precision,
                           preferred_element_type=jnp.float32)


def _sc_mesh():
    return plsc.VectorSubcoreMesh(core_axis_name="c", subcore_axis_name="s")


def _sc_gather(table, idx):
    e = idx.shape[0]
    d = table.shape[1]
    idx2 = idx.reshape(1, e)

    @functools.partial(
        pl.kernel,
        out_type=jax.ShapeDtypeStruct((e, d), table.dtype),
        mesh=_sc_mesh(),
        compiler_params=pltpu.CompilerParams(use_tc_tiling_on_sc=False),
    )
    def k(tab_hbm, i_hbm, o_hbm):
        def body(i_vmem, o_vmem):
            pltpu.sync_copy(tab_hbm.at[i_vmem.at[0]], o_vmem)

        pltpu.emit_pipeline(
            body,
            grid=(e // _W,),
            in_specs=[pl.BlockSpec((1, _W), lambda i: (0, i))],
            out_specs=[pl.BlockSpec((_W, d), lambda i: (i, 0))],
            core_axis_name=("c", "s"),
            dimension_semantics=(pltpu.PARALLEL,),
        )(i_hbm, o_hbm)

    return k(table, idx2)


def _sc_scatter_add(vals, idx, n_rows, zeros):
    e, d = vals.shape
    idx2 = idx.reshape(1, e)
    per_s = n_rows // 16

    @functools.partial(
        pl.kernel,
        out_type=jax.ShapeDtypeStruct((2, n_rows, d), vals.dtype),
        mesh=_sc_mesh(),
        scratch_types=[pltpu.VMEM_SHARED((n_rows, d), jnp.float32)],
        compiler_params=pltpu.CompilerParams(use_tc_tiling_on_sc=False),
    )
    def k(v_hbm, i_hbm, z_hbm, o_hbm, acc_sh):
        cid = lax.axis_index("c")
        sid = lax.axis_index("s")
        sl = pl.ds(sid * per_s, per_s)
        pltpu.sync_copy(z_hbm.at[sl], acc_sh.at[sl])
        plsc.subcore_barrier()

        def body(v_vmem, i_vmem):
            pltpu.sync_copy(v_vmem, acc_sh.at[i_vmem.at[0]], add=True)

        pltpu.emit_pipeline(
            body,
            grid=(e // _W,),
            in_specs=[pl.BlockSpec((_W, d), lambda i: (i, 0)),
                      pl.BlockSpec((1, _W), lambda i: (0, i))],
            out_specs=[],
            core_axis_name=("c", "s"),
            dimension_semantics=(pltpu.PARALLEL,),
        )(v_hbm, i_hbm)
        plsc.subcore_barrier()
        pltpu.sync_copy(acc_sh.at[sl], o_hbm.at[cid].at[sl])

    return k(vals, idx2, zeros)


def _full(spec_arr):
    nd = spec_arr.ndim
    return pl.BlockSpec(spec_arr.shape, lambda i: (0,) * nd)


def _node_tc(x, W_node, b_node, bn=2000):
    n = x.shape[0]

    def body(x_ref, w_ref, b_ref, o_ref):
        o_ref[...] = jnp.maximum(
            _mm(x_ref[...], w_ref[...], lax.Precision.DEFAULT) + b_ref[...],
            0.0)

    return pl.pallas_call(
        body,
        grid=(n // bn,),
        in_specs=[pl.BlockSpec((bn, x.shape[1]), lambda i: (i, 0)),
                  _full(W_node), _full(b_node)],
        out_specs=pl.BlockSpec((bn, D), lambda i: (i, 0)),
        out_shape=jax.ShapeDtypeStruct((n, D), jnp.float32),
    )(x, W_node, b_node)


def _msg_tc(edge_attr, g, W_ea, b_ea, W_nn1, b_nn1, W_nn2, b_nn2, rep,
            be=3200):
    e = edge_attr.shape[0]

    def body(ea_ref, g_ref, wea, bea, w1, b1, w2, b2, rep_ref, o_ref):
        fast = lax.Precision.DEFAULT
        ea = jnp.maximum(_mm(ea_ref[...], wea[...], fast) + bea[...], 0.0)
        r = jnp.maximum(_mm(ea, w1[...], fast) + b1[...], 0.0)
        w = _mm(r, w2[...], fast) + b2[...]
        gexp = _mm(g_ref[...], rep_ref[...], fast)
        t = gexp * w
        a = t[:, :512] + t[:, 512:]
        a = a[:, :256] + a[:, 256:]
        a = a[:, :128] + a[:, 128:]
        a = a[:, :64] + a[:, 64:]
        o_ref[...] = a[:, :32] + a[:, 32:]

    return pl.pallas_call(
        body,
        grid=(e // be,),
        in_specs=[pl.BlockSpec((be, edge_attr.shape[1]), lambda i: (i, 0)),
                  pl.BlockSpec((be, D), lambda i: (i, 0)),
                  _full(W_ea), _full(b_ea), _full(W_nn1), _full(b_nn1),
                  _full(W_nn2), _full(b_nn2), _full(rep)],
        out_specs=pl.BlockSpec((be, D), lambda i: (i, 0)),
        out_shape=jax.ShapeDtypeStruct((e, D), jnp.float32),
    )(edge_attr, g, W_ea, b_ea, W_nn1, b_nn1, W_nn2, b_nn2, rep)


def _gru_tc(a0, a1, d0, d1, h, b_conv, W_ih, b_ih, W_hh, b_hh, bn=2000):
    n = h.shape[0]

    def body(a0r, a1r, d0r, d1r, hr, bc, wih, bih, whh, bhh, o_ref):
        deg = jnp.maximum(d0r[...][:, 0:1] + d1r[...][:, 0:1], 1.0)
        agg = (a0r[...] + a1r[...]) / deg
        m = jnp.maximum(agg + bc[...], 0.0)
        fast = lax.Precision.DEFAULT
        gi = _mm(m, wih[...], fast) + bih[...]
        gh = _mm(hr[...], whh[...], fast) + bhh[...]
        r = jax.nn.sigmoid(gi[:, :D] + gh[:, :D])
        z = jax.nn.sigmoid(gi[:, D:2 * D] + gh[:, D:2 * D])
        nn = jnp.tanh(gi[:, 2 * D:] + r * gh[:, 2 * D:])
        o_ref[...] = (1.0 - z) * nn + z * hr[...]

    return pl.pallas_call(
        body,
        grid=(n // bn,),
        in_specs=[pl.BlockSpec((bn, D), lambda i: (i, 0)),
                  pl.BlockSpec((bn, D), lambda i: (i, 0)),
                  pl.BlockSpec((bn, 16), lambda i: (i, 0)),
                  pl.BlockSpec((bn, 16), lambda i: (i, 0)),
                  pl.BlockSpec((bn, D), lambda i: (i, 0)),
                  _full(b_conv), _full(W_ih), _full(b_ih), _full(W_hh),
                  _full(b_hh)],
        out_specs=pl.BlockSpec((bn, D), lambda i: (i, 0)),
        out_shape=jax.ShapeDtypeStruct((n, D), jnp.float32),
    )(a0, a1, d0, d1, h, b_conv, W_ih, b_ih, W_hh, b_hh)


def _feat(t0, t1):
    return jnp.concatenate([(t0 + t1) * 0.5, t0 * t1, (t0 - t1) ** 2], axis=1)


def _readout_stats_tc(t01, e, be=4000):
    nb = e // be

    def body(t0r, t1r, s_ref, q_ref):
        f = _feat(t0r[...], t1r[...])

        @pl.when(pl.program_id(0) == 0)
        def _():
            s_ref[...] = jnp.zeros_like(s_ref)
            q_ref[...] = jnp.zeros_like(q_ref)

        s_ref[...] += jnp.sum(f, axis=0, keepdims=True)
        q_ref[...] += jnp.sum(f * f, axis=0, keepdims=True)

    return pl.pallas_call(
        body,
        grid=(nb,),
        in_specs=[pl.BlockSpec((be, D), lambda i: (i, 0)),
                  pl.BlockSpec((be, D), lambda i: (i + nb, 0))],
        out_specs=[pl.BlockSpec((1, 3 * D), lambda i: (0, 0)),
                   pl.BlockSpec((1, 3 * D), lambda i: (0, 0))],
        out_shape=[jax.ShapeDtypeStruct((1, 3 * D), jnp.float32),
                   jax.ShapeDtypeStruct((1, 3 * D), jnp.float32)],
    )(t01, t01)


def _readout_tc(t01, ea3, s, q, gamma, beta, W_w, W_b, be=4000):
    e = ea3.shape[0]
    nb = e // be
    inv_e = 1.0 / float(e)

    def body(t0r, t1r, ea3r, s_ref, q_ref, g_ref, b_ref, ww, wb, o_ref):
        f = _feat(t0r[...], t1r[...])
        mean = s_ref[...] * inv_e
        var = q_ref[...] * inv_e - mean * mean
        scale = g_ref[...] * lax.rsqrt(var + 1e-5)
        shift = b_ref[...] - mean * scale
        wgt = _mm(ea3r[...], ww[...], lax.Precision.DEFAULT)
        bias = _mm(ea3r[...], wb[...], lax.Precision.DEFAULT)
        fn = f * scale + shift
        o_ref[...] = jnp.sum(fn * wgt, axis=1, keepdims=True) + bias

    return pl.pallas_call(
        body,
        grid=(nb,),
        in_specs=[pl.BlockSpec((be, D), lambda i: (i, 0)),
                  pl.BlockSpec((be, D), lambda i: (i + nb, 0)),
                  pl.BlockSpec((be, ea3.shape[1]), lambda i: (i, 0)),
                  _full(s), _full(q), _full(gamma), _full(beta),
                  _full(W_w), _full(W_b)],
        out_specs=pl.BlockSpec((be, 1), lambda i: (i, 0)),
        out_shape=jax.ShapeDtypeStruct((e, 1), jnp.float32),
    )(t01, t01, ea3, s, q, gamma, beta, W_w, W_b)


def kernel(x, edge_attr, edge_attr3, W_node, b_node, W_ea, b_ea, W_nn1, b_nn1,
           W_nn2, b_nn2, b_conv, W_ih, b_ih, W_hh, b_hh, gamma, beta, W_w, W_b,
           edge_index, edge_index3):
    n = x.shape[0]
    e = edge_index.shape[1]
    e3 = edge_index3.shape[1]

    src = edge_index[0].astype(jnp.int32)
    dst = edge_index[1].astype(jnp.int32)
    i3a = edge_index3[0].astype(jnp.int32)
    i3b = edge_index3[1].astype(jnp.int32)

    b_node2 = b_node.reshape(1, -1)
    b_ea2 = b_ea.reshape(1, -1)
    b_nn12 = b_nn1.reshape(1, -1)
    b_nn22 = b_nn2.reshape(1, -1)
    b_conv2 = b_conv.reshape(1, -1)
    b_ih2 = b_ih.reshape(1, -1)
    b_hh2 = b_hh.reshape(1, -1)
    gamma2 = gamma.reshape(1, -1)
    beta2 = beta.reshape(1, -1)

    ll = jnp.arange(D * D)
    rep = (ll[None, :] // D == jnp.arange(D)[:, None]).astype(jnp.float32)

    zeros_nd = jnp.zeros((n, D), jnp.float32)
    zeros_n16 = jnp.zeros((n, 16), jnp.float32)
    ones_e16 = jnp.ones((e, 16), jnp.float32)

    h = _node_tc(x, W_node, b_node2)

    degp = _sc_scatter_add(ones_e16, dst, n, zeros_n16)
    d0, d1 = degp[0], degp[1]

    for _ in range(3):
        g = _sc_gather(h, src)
        msg = _msg_tc(edge_attr, g, W_ea, b_ea2, W_nn1, b_nn12, W_nn2, b_nn22,
                      rep)
        p = _sc_scatter_add(msg, dst, n, zeros_nd)
        h = _gru_tc(p[0], p[1], d0, d1, h, b_conv2, W_ih, b_ih2, W_hh, b_hh2)

    t01 = _sc_gather(h, jnp.concatenate([i3a, i3b]))
    s, q = _readout_stats_tc(t01, e3)
    yhat = _readout_tc(t01, edge_attr3, s, q, gamma2, beta2, W_w, W_b)
    return yhat.reshape(e3)


# speedup vs baseline: 1.0383x; 1.0083x over previous
import functools

import jax
import jax.numpy as jnp
from jax import lax
from jax.experimental import pallas as pl
from jax.experimental.pallas import tpu as pltpu
from jax.experimental.pallas import tpu_sc as plsc

D = 32
_W = 128


def _mm(a, b, precision=lax.Precision.HIGHEST):
    return lax.dot_general(a, b, (((1,), (0,)), ((), ())),
                           precision=precision,
                           preferred_element_type=jnp.float32)


def _sc_mesh():
    return plsc.VectorSubcoreMesh(core_axis_name="c", subcore_axis_name="s")


def _sc_gather(table, idx):
    e = idx.shape[0]
    d = table.shape[1]
    idx2 = idx.reshape(1, e)

    @functools.partial(
        pl.kernel,
        out_type=jax.ShapeDtypeStruct((e, d), table.dtype),
        mesh=_sc_mesh(),
        compiler_params=pltpu.CompilerParams(use_tc_tiling_on_sc=False),
    )
    def k(tab_hbm, i_hbm, o_hbm):
        def body(i_vmem, o_vmem):
            pltpu.sync_copy(tab_hbm.at[i_vmem.at[0]], o_vmem)

        pltpu.emit_pipeline(
            body,
            grid=(e // _W,),
            in_specs=[pl.BlockSpec((1, _W), lambda i: (0, i))],
            out_specs=[pl.BlockSpec((_W, d), lambda i: (i, 0))],
            core_axis_name=("c", "s"),
            dimension_semantics=(pltpu.PARALLEL,),
        )(i_hbm, o_hbm)

    return k(table, idx2)


def _sc_scatter_add(vals, idx, n_rows, zeros):
    e, d = vals.shape
    idx2 = idx.reshape(1, e)
    per_s = n_rows // 16

    @functools.partial(
        pl.kernel,
        out_type=jax.ShapeDtypeStruct((2, n_rows, d), vals.dtype),
        mesh=_sc_mesh(),
        scratch_types=[pltpu.VMEM_SHARED((n_rows, d), jnp.float32)],
        compiler_params=pltpu.CompilerParams(use_tc_tiling_on_sc=False),
    )
    def k(v_hbm, i_hbm, z_hbm, o_hbm, acc_sh):
        cid = lax.axis_index("c")
        sid = lax.axis_index("s")
        sl = pl.ds(sid * per_s, per_s)
        pltpu.sync_copy(z_hbm.at[sl], acc_sh.at[sl])
        plsc.subcore_barrier()

        def body(v_vmem, i_vmem):
            pltpu.sync_copy(v_vmem, acc_sh.at[i_vmem.at[0]], add=True)

        pltpu.emit_pipeline(
            body,
            grid=(e // _W,),
            in_specs=[pl.BlockSpec((_W, d), lambda i: (i, 0)),
                      pl.BlockSpec((1, _W), lambda i: (0, i))],
            out_specs=[],
            core_axis_name=("c", "s"),
            dimension_semantics=(pltpu.PARALLEL,),
        )(v_hbm, i_hbm)
        plsc.subcore_barrier()
        pltpu.sync_copy(acc_sh.at[sl], o_hbm.at[cid].at[sl])

    return k(vals, idx2, zeros)


def _full(spec_arr):
    nd = spec_arr.ndim
    return pl.BlockSpec(spec_arr.shape, lambda i: (0,) * nd)


def _node_tc(x, W_node, b_node, bn=2000):
    n = x.shape[0]

    def body(x_ref, w_ref, b_ref, o_ref):
        o_ref[...] = jnp.maximum(
            _mm(x_ref[...], w_ref[...], lax.Precision.DEFAULT) + b_ref[...],
            0.0)

    return pl.pallas_call(
        body,
        grid=(n // bn,),
        in_specs=[pl.BlockSpec((bn, x.shape[1]), lambda i: (i, 0)),
                  _full(W_node), _full(b_node)],
        out_specs=pl.BlockSpec((bn, D), lambda i: (i, 0)),
        out_shape=jax.ShapeDtypeStruct((n, D), jnp.float32),
    )(x, W_node, b_node)


def _msg_tc(edge_attr, g, W_ea, b_ea, W_nn1, b_nn1, W_nn2, b_nn2, rep,
            be=6400):
    e = edge_attr.shape[0]

    def body(ea_ref, g_ref, wea, bea, w1, b1, w2, b2, rep_ref, o_ref):
        fast = lax.Precision.DEFAULT
        ea = jnp.maximum(_mm(ea_ref[...], wea[...], fast) + bea[...], 0.0)
        r = jnp.maximum(_mm(ea, w1[...], fast) + b1[...], 0.0)
        w = _mm(r, w2[...], fast) + b2[...]
        gexp = _mm(g_ref[...], rep_ref[...], fast)
        t = gexp * w
        a = t[:, :512] + t[:, 512:]
        a = a[:, :256] + a[:, 256:]
        a = a[:, :128] + a[:, 128:]
        a = a[:, :64] + a[:, 64:]
        o_ref[...] = a[:, :32] + a[:, 32:]

    return pl.pallas_call(
        body,
        grid=(e // be,),
        in_specs=[pl.BlockSpec((be, edge_attr.shape[1]), lambda i: (i, 0)),
                  pl.BlockSpec((be, D), lambda i: (i, 0)),
                  _full(W_ea), _full(b_ea), _full(W_nn1), _full(b_nn1),
                  _full(W_nn2), _full(b_nn2), _full(rep)],
        out_specs=pl.BlockSpec((be, D), lambda i: (i, 0)),
        out_shape=jax.ShapeDtypeStruct((e, D), jnp.float32),
    )(edge_attr, g, W_ea, b_ea, W_nn1, b_nn1, W_nn2, b_nn2, rep)


def _gru_tc(a0, a1, d0, d1, h, b_conv, W_ih, b_ih, W_hh, b_hh, bn=2000):
    n = h.shape[0]

    def body(a0r, a1r, d0r, d1r, hr, bc, wih, bih, whh, bhh, o_ref):
        deg = jnp.maximum(d0r[...][:, 0:1] + d1r[...][:, 0:1], 1.0)
        agg = (a0r[...] + a1r[...]) / deg
        m = jnp.maximum(agg + bc[...], 0.0)
        fast = lax.Precision.DEFAULT
        gi = _mm(m, wih[...], fast) + bih[...]
        gh = _mm(hr[...], whh[...], fast) + bhh[...]
        r = jax.nn.sigmoid(gi[:, :D] + gh[:, :D])
        z = jax.nn.sigmoid(gi[:, D:2 * D] + gh[:, D:2 * D])
        nn = jnp.tanh(gi[:, 2 * D:] + r * gh[:, 2 * D:])
        o_ref[...] = (1.0 - z) * nn + z * hr[...]

    return pl.pallas_call(
        body,
        grid=(n // bn,),
        in_specs=[pl.BlockSpec((bn, D), lambda i: (i, 0)),
                  pl.BlockSpec((bn, D), lambda i: (i, 0)),
                  pl.BlockSpec((bn, 16), lambda i: (i, 0)),
                  pl.BlockSpec((bn, 16), lambda i: (i, 0)),
                  pl.BlockSpec((bn, D), lambda i: (i, 0)),
                  _full(b_conv), _full(W_ih), _full(b_ih), _full(W_hh),
                  _full(b_hh)],
        out_specs=pl.BlockSpec((bn, D), lambda i: (i, 0)),
        out_shape=jax.ShapeDtypeStruct((n, D), jnp.float32),
    )(a0, a1, d0, d1, h, b_conv, W_ih, b_ih, W_hh, b_hh)


def _feat(t0, t1):
    return jnp.concatenate([(t0 + t1) * 0.5, t0 * t1, (t0 - t1) ** 2], axis=1)


def _readout_stats_tc(t01, e, be=4000):
    nb = e // be

    def body(t0r, t1r, s_ref, q_ref):
        f = _feat(t0r[...], t1r[...])

        @pl.when(pl.program_id(0) == 0)
        def _():
            s_ref[...] = jnp.zeros_like(s_ref)
            q_ref[...] = jnp.zeros_like(q_ref)

        s_ref[...] += jnp.sum(f, axis=0, keepdims=True)
        q_ref[...] += jnp.sum(f * f, axis=0, keepdims=True)

    return pl.pallas_call(
        body,
        grid=(nb,),
        in_specs=[pl.BlockSpec((be, D), lambda i: (i, 0)),
                  pl.BlockSpec((be, D), lambda i: (i + nb, 0))],
        out_specs=[pl.BlockSpec((1, 3 * D), lambda i: (0, 0)),
                   pl.BlockSpec((1, 3 * D), lambda i: (0, 0))],
        out_shape=[jax.ShapeDtypeStruct((1, 3 * D), jnp.float32),
                   jax.ShapeDtypeStruct((1, 3 * D), jnp.float32)],
    )(t01, t01)


def _readout_tc(t01, ea3, s, q, gamma, beta, W_w, W_b, be=4000):
    e = ea3.shape[0]
    nb = e // be
    inv_e = 1.0 / float(e)

    def body(t0r, t1r, ea3r, s_ref, q_ref, g_ref, b_ref, ww, wb, o_ref):
        f = _feat(t0r[...], t1r[...])
        mean = s_ref[...] * inv_e
        var = q_ref[...] * inv_e - mean * mean
        scale = g_ref[...] * lax.rsqrt(var + 1e-5)
        shift = b_ref[...] - mean * scale
        wgt = _mm(ea3r[...], ww[...], lax.Precision.DEFAULT)
        bias = _mm(ea3r[...], wb[...], lax.Precision.DEFAULT)
        fn = f * scale + shift
        o_ref[...] = jnp.sum(fn * wgt, axis=1, keepdims=True) + bias

    return pl.pallas_call(
        body,
        grid=(nb,),
        in_specs=[pl.BlockSpec((be, D), lambda i: (i, 0)),
                  pl.BlockSpec((be, D), lambda i: (i + nb, 0)),
                  pl.BlockSpec((be, ea3.shape[1]), lambda i: (i, 0)),
                  _full(s), _full(q), _full(gamma), _full(beta),
                  _full(W_w), _full(W_b)],
        out_specs=pl.BlockSpec((be, 1), lambda i: (i, 0)),
        out_shape=jax.ShapeDtypeStruct((e, 1), jnp.float32),
    )(t01, t01, ea3, s, q, gamma, beta, W_w, W_b)


def kernel(x, edge_attr, edge_attr3, W_node, b_node, W_ea, b_ea, W_nn1, b_nn1,
           W_nn2, b_nn2, b_conv, W_ih, b_ih, W_hh, b_hh, gamma, beta, W_w, W_b,
           edge_index, edge_index3):
    n = x.shape[0]
    e = edge_index.shape[1]
    e3 = edge_index3.shape[1]

    src = edge_index[0].astype(jnp.int32)
    dst = edge_index[1].astype(jnp.int32)
    i3a = edge_index3[0].astype(jnp.int32)
    i3b = edge_index3[1].astype(jnp.int32)

    b_node2 = b_node.reshape(1, -1)
    b_ea2 = b_ea.reshape(1, -1)
    b_nn12 = b_nn1.reshape(1, -1)
    b_nn22 = b_nn2.reshape(1, -1)
    b_conv2 = b_conv.reshape(1, -1)
    b_ih2 = b_ih.reshape(1, -1)
    b_hh2 = b_hh.reshape(1, -1)
    gamma2 = gamma.reshape(1, -1)
    beta2 = beta.reshape(1, -1)

    ll = jnp.arange(D * D)
    rep = (ll[None, :] // D == jnp.arange(D)[:, None]).astype(jnp.float32)

    zeros_nd = jnp.zeros((n, D), jnp.float32)
    zeros_n16 = jnp.zeros((n, 16), jnp.float32)
    ones_e16 = jnp.ones((e, 16), jnp.float32)

    h = _node_tc(x, W_node, b_node2)

    degp = _sc_scatter_add(ones_e16, dst, n, zeros_n16)
    d0, d1 = degp[0], degp[1]

    for _ in range(3):
        g = _sc_gather(h, src)
        msg = _msg_tc(edge_attr, g, W_ea, b_ea2, W_nn1, b_nn12, W_nn2, b_nn22,
                      rep)
        p = _sc_scatter_add(msg, dst, n, zeros_nd)
        h = _gru_tc(p[0], p[1], d0, d1, h, b_conv2, W_ih, b_ih2, W_hh, b_hh2)

    t01 = _sc_gather(h, jnp.concatenate([i3a, i3b]))
    s, q = _readout_stats_tc(t01, e3)
    yhat = _readout_tc(t01, edge_attr3, s, q, gamma2, beta2, W_w, W_b)
    return yhat.reshape(e3)
